# baseline (speedup 1.0000x reference)
import functools

import jax
import jax.numpy as jnp
from jax import lax
from jax.experimental import pallas as pl
from jax.experimental.pallas import tpu as pltpu
from jax.experimental.pallas import tpu_sc as plsc

N = 10000
NP = 10240
D = 128
E = 320000
CLS = 40
NC = 2
NS = 16
NW = NC * NS
EW = E // NW
CHUNK = 80
NCH = EW // CHUNK
RPT = NP // NS
ZR = 128

BN = 1024
GRID = NP // BN

_mesh = plsc.VectorSubcoreMesh(core_axis_name="c", subcore_axis_name="s")


def _fill_zero(ref, rows, width):
    @pl.loop(0, rows)
    def _(r):
        @pl.loop(0, width, step=16)
        def _(cc):
            ref.at[pl.ds(r, 1), pl.ds(cc, 16)][...] = jnp.zeros((1, 16), jnp.float32)


def _prop_body(g_hbm, ei_hbm, out_hbm, idx_v, rows_v, acc_sh):
    c = lax.axis_index("c")
    s = lax.axis_index("s")
    wid = c * NS + s

    _fill_zero(rows_v, CHUNK, D)

    @pl.loop(0, RPT // CHUNK)
    def _(k):
        pltpu.sync_copy(rows_v, acc_sh.at[pl.ds(s * RPT + k * CHUNK, CHUNK)])

    pltpu.sync_copy(ei_hbm.at[wid], idx_v)
    plsc.subcore_barrier()

    @pl.loop(0, NCH)
    def _(j):
        pltpu.sync_copy(g_hbm.at[idx_v.at[j]], rows_v)
        pltpu.sync_copy(rows_v, acc_sh.at[idx_v.at[NCH + j]], add=True)

    plsc.subcore_barrier()
    pltpu.sync_copy(acc_sh.at[pl.ds(s * RPT, RPT)],
                    out_hbm.at[c, pl.ds(s * RPT, RPT)])


def _sc_prop(g, eim):
    return pl.kernel(
        _prop_body,
        out_type=jax.ShapeDtypeStruct((NC, NP, D), jnp.float32),
        mesh=_mesh,
        scratch_types=[
            pltpu.VMEM((2 * NCH, CHUNK), jnp.int32),
            pltpu.VMEM((CHUNK, D), jnp.float32),
            pltpu.VMEM_SHARED((NP, D), jnp.float32),
        ],
    )(g, eim)


def _tc_dis_g0(deg_part, x):
    def body(dp, xr, dis_ref, g0_ref):
        a = dp[...]
        d = a[0, :, 0:1] + a[1, :, 0:1]
        dis1 = jnp.where(d > 0, lax.rsqrt(d), 0.0)
        dis_ref[...] = jnp.broadcast_to(dis1, (BN, 16))
        g0_ref[...] = xr[...] * dis1

    return pl.pallas_call(
        body,
        grid=(GRID,),
        in_specs=[pl.BlockSpec((NC, BN, D), lambda j: (0, j, 0)),
                  pl.BlockSpec((BN, D), lambda j: (j, 0))],
        out_specs=[pl.BlockSpec((BN, 16), lambda j: (j, 0)),
                   pl.BlockSpec((BN, D), lambda j: (j, 0))],
        out_shape=[jax.ShapeDtypeStruct((NP, 16), jnp.float32),
                   jax.ShapeDtypeStruct((NP, D), jnp.float32)],
    )(deg_part, x)


def _tc_fuse(p, dis):
    def body(pr, dr, u_ref, g_ref):
        d1 = dr[...][:, 0:1]
        a = pr[...]
        u = -d1 * (a[0] + a[1])
        u_ref[...] = u
        g_ref[...] = d1 * u

    return pl.pallas_call(
        body,
        grid=(GRID,),
        in_specs=[pl.BlockSpec((NC, BN, D), lambda j: (0, j, 0)),
                  pl.BlockSpec((BN, 16), lambda j: (j, 0))],
        out_specs=[pl.BlockSpec((BN, D), lambda j: (j, 0)),
                   pl.BlockSpec((BN, D), lambda j: (j, 0))],
        out_shape=[jax.ShapeDtypeStruct((NP, D), jnp.float32),
                   jax.ShapeDtypeStruct((NP, D), jnp.float32)],
    )(p, dis)


def _tc_mm1(u0, u1, p, dis, W, b):
    def body(u0r, u1r, pr, dr, wr, br, h_ref, g_ref):
        d1 = dr[...][:, 0:1]
        a = pr[...]
        u0v = u0r[...]
        u2 = -2.0 * d1 * (a[0] + a[1]) - u0v
        w = wr[...]
        acc = (jnp.dot(u0v, w[0], preferred_element_type=jnp.float32,
                       precision=lax.Precision.HIGHEST)
               + jnp.dot(u1r[...], w[1], preferred_element_type=jnp.float32,
                         precision=lax.Precision.HIGHEST)
               + jnp.dot(u2, w[2], preferred_element_type=jnp.float32,
                         precision=lax.Precision.HIGHEST)
               + br[...])
        h = jnp.maximum(acc, 0.0)
        h_ref[...] = h
        g_ref[...] = d1 * h

    return pl.pallas_call(
        body,
        grid=(GRID,),
        in_specs=[pl.BlockSpec((BN, D), lambda j: (j, 0)),
                  pl.BlockSpec((BN, D), lambda j: (j, 0)),
                  pl.BlockSpec((NC, BN, D), lambda j: (0, j, 0)),
                  pl.BlockSpec((BN, 16), lambda j: (j, 0)),
                  pl.BlockSpec((3, D, D), lambda j: (0, 0, 0)),
                  pl.BlockSpec((1, D), lambda j: (0, 0))],
        out_specs=[pl.BlockSpec((BN, D), lambda j: (j, 0)),
                   pl.BlockSpec((BN, D), lambda j: (j, 0))],
        out_shape=[jax.ShapeDtypeStruct((NP, D), jnp.float32),
                   jax.ShapeDtypeStruct((NP, D), jnp.float32)],
    )(u0, u1, p, dis, W, b)


def _tc_mm2(u0, u1, p, dis, W, b):
    def body(u0r, u1r, pr, dr, wr, br, o_ref):
        d1 = dr[...][:, 0:1]
        a = pr[...]
        u0v = u0r[...]
        u2 = -2.0 * d1 * (a[0] + a[1]) - u0v
        w = wr[...]
        acc = (jnp.dot(u0v, w[0], preferred_element_type=jnp.float32,
                       precision=lax.Precision.HIGHEST)
               + jnp.dot(u1r[...], w[1], preferred_element_type=jnp.float32,
                         precision=lax.Precision.HIGHEST)
               + jnp.dot(u2, w[2], preferred_element_type=jnp.float32,
                         precision=lax.Precision.HIGHEST)
               + br[...])
        m = jnp.max(acc, axis=-1, keepdims=True)
        sh = acc - m
        lse = jnp.log(jnp.sum(jnp.exp(sh), axis=-1, keepdims=True))
        o_ref[...] = sh - lse

    return pl.pallas_call(
        body,
        grid=(GRID,),
        in_specs=[pl.BlockSpec((BN, D), lambda j: (j, 0)),
                  pl.BlockSpec((BN, D), lambda j: (j, 0)),
                  pl.BlockSpec((NC, BN, D), lambda j: (0, j, 0)),
                  pl.BlockSpec((BN, 16), lambda j: (j, 0)),
                  pl.BlockSpec((3, D, CLS), lambda j: (0, 0, 0)),
                  pl.BlockSpec((1, CLS), lambda j: (0, 0))],
        out_specs=pl.BlockSpec((BN, CLS), lambda j: (j, 0)),
        out_shape=jax.ShapeDtypeStruct((NP, CLS), jnp.float32),
    )(u0, u1, p, dis, W, b)


def kernel(x, edge_index, W1, b1, W2, b2):
    xp = jnp.pad(x, ((0, NP - N), (0, 0)))
    rowm = edge_index[0].reshape(NW, NCH, CHUNK)
    colm = edge_index[1].reshape(NW, NCH, CHUNK)
    eim = jnp.concatenate([rowm, colm], axis=1)
    b1r = b1.reshape(1, D)
    b2r = b2.reshape(1, CLS)

    ones_tbl = jnp.ones((8, D), jnp.float32)
    eim_deg = jnp.concatenate([jnp.zeros_like(rowm), rowm], axis=1)
    deg_part = _sc_prop(ones_tbl, eim_deg)
    dis, g0 = _tc_dis_g0(deg_part, xp)

    p1 = _sc_prop(g0, eim)
    u1, g1 = _tc_fuse(p1, dis)
    p2 = _sc_prop(g1, eim)
    h, g2 = _tc_mm1(xp, u1, p2, dis, W1, b1r)

    p3 = _sc_prop(g2, eim)
    v1, g3 = _tc_fuse(p3, dis)
    p4 = _sc_prop(g3, eim)
    return _tc_mm2(h, v1, p4, dis, W2, b2r)[:N]


# speedup vs baseline: 1.9382x; 1.9382x over previous
import functools

import jax
import jax.numpy as jnp
from jax import lax
from jax.experimental import pallas as pl
from jax.experimental.pallas import tpu as pltpu
from jax.experimental.pallas import tpu_sc as plsc

N = 10000
NP = 10240
D = 128
DH = 64
E = 320000
CLS = 40
NC = 2
NS = 16
EW = E // NS
CHUNK = 80
NCH = EW // CHUNK
RPT = NP // NS
NBUF = 5
LOOK = 2

BN = 1024
GRID = NP // BN

_mesh = plsc.VectorSubcoreMesh(core_axis_name="c", subcore_axis_name="s")


def _fill_zero(ref, rows, width):
    @pl.loop(0, rows)
    def _(r):
        @pl.loop(0, width, step=16)
        def _(cc):
            ref.at[pl.ds(r, 1), pl.ds(cc, 16)][...] = jnp.zeros((1, 16), jnp.float32)


def _prop_body(g2_hbm, ei_hbm, out_hbm, idx_v, rows_v, acc_sh, sem_i, sem_g, sem_s):
    c = lax.axis_index("c")
    s = lax.axis_index("s")

    _fill_zero(rows_v.at[0], CHUNK, DH)

    @pl.loop(0, RPT // CHUNK)
    def _(k):
        pltpu.sync_copy(rows_v.at[0], acc_sh.at[pl.ds(s * RPT + k * CHUNK, CHUNK)])

    plsc.subcore_barrier()

    for b in range(NBUF):
        pltpu.async_copy(ei_hbm.at[s, b], idx_v.at[b], sem_i.at[b])

    @pl.loop(0, NCH // NBUF)
    def _(o):
        j0 = o * NBUF
        for b in range(NBUF):
            j = j0 + b
            pltpu.make_async_copy(ei_hbm.at[s, 0], idx_v.at[b],
                                  sem_i.at[b]).wait()
            pltpu.async_copy(g2_hbm.at[c].at[idx_v.at[b].at[0]], rows_v.at[b],
                             sem_g.at[b])

            b2 = (b - LOOK) % NBUF

            @pl.when(j >= LOOK)
            def _():
                pltpu.make_async_copy(g2_hbm.at[0].at[idx_v.at[0].at[0]],
                                      rows_v.at[b2], sem_g.at[b2]).wait()
                pltpu.async_copy(rows_v.at[b2], acc_sh.at[idx_v.at[b2].at[1]],
                                 sem_s.at[b2], add=True)
                pltpu.make_async_copy(rows_v.at[b2],
                                      acc_sh.at[idx_v.at[b2].at[1]],
                                      sem_s.at[b2]).wait()

                @pl.when(j - LOOK + NBUF < NCH)
                def _():
                    pltpu.async_copy(ei_hbm.at[s, j - LOOK + NBUF],
                                     idx_v.at[b2], sem_i.at[b2])

    for k in range(NCH - LOOK, NCH):
        b2 = k % NBUF
        pltpu.make_async_copy(g2_hbm.at[0].at[idx_v.at[0].at[0]],
                              rows_v.at[b2], sem_g.at[b2]).wait()
        pltpu.async_copy(rows_v.at[b2], acc_sh.at[idx_v.at[b2].at[1]],
                         sem_s.at[b2], add=True)
        pltpu.make_async_copy(rows_v.at[b2], acc_sh.at[idx_v.at[b2].at[1]],
                              sem_s.at[b2]).wait()

    plsc.subcore_barrier()
    pltpu.sync_copy(acc_sh.at[pl.ds(s * RPT, RPT)],
                    out_hbm.at[c, pl.ds(s * RPT, RPT)])


def _sc_prop(g2, ei):
    return pl.kernel(
        _prop_body,
        out_type=jax.ShapeDtypeStruct((NC, NP, DH), jnp.float32),
        mesh=_mesh,
        compiler_params=pltpu.CompilerParams(use_tc_tiling_on_sc=False),
        scratch_types=[
            pltpu.VMEM((NBUF, 2, CHUNK), jnp.int32),
            pltpu.VMEM((NBUF, CHUNK, DH), jnp.float32),
            pltpu.VMEM_SHARED((NP, DH), jnp.float32),
            pltpu.SemaphoreType.DMA((NBUF,)),
            pltpu.SemaphoreType.DMA((NBUF,)),
            pltpu.SemaphoreType.DMA((NBUF,)),
        ],
    )(g2, ei)


def _tc_dis_g0(deg_part, x):
    def body(dp, xr, dis_ref, g0_ref):
        d = dp[...][0, :, 0:1]
        dis1 = jnp.where(d > 0, lax.rsqrt(d), 0.0)
        dis_ref[...] = jnp.broadcast_to(dis1, (BN, 16))
        xv = xr[...]
        g0_ref[0] = xv[:, :DH] * dis1
        g0_ref[1] = xv[:, DH:] * dis1

    return pl.pallas_call(
        body,
        grid=(GRID,),
        in_specs=[pl.BlockSpec((NC, BN, DH), lambda j: (0, j, 0)),
                  pl.BlockSpec((BN, D), lambda j: (j, 0))],
        out_specs=[pl.BlockSpec((BN, 16), lambda j: (j, 0)),
                   pl.BlockSpec((NC, BN, DH), lambda j: (0, j, 0))],
        out_shape=[jax.ShapeDtypeStruct((NP, 16), jnp.float32),
                   jax.ShapeDtypeStruct((NC, NP, DH), jnp.float32)],
    )(deg_part, x)


def _tc_fuse(p, dis):
    def body(pr, dr, u_ref, g_ref):
        d1 = dr[...][:, 0:1]
        a = pr[...]
        pfull = jnp.concatenate([a[0], a[1]], axis=-1)
        u = -d1 * pfull
        u_ref[...] = u
        g = d1 * u
        g_ref[0] = g[:, :DH]
        g_ref[1] = g[:, DH:]

    return pl.pallas_call(
        body,
        grid=(GRID,),
        in_specs=[pl.BlockSpec((NC, BN, DH), lambda j: (0, j, 0)),
                  pl.BlockSpec((BN, 16), lambda j: (j, 0))],
        out_specs=[pl.BlockSpec((BN, D), lambda j: (j, 0)),
                   pl.BlockSpec((NC, BN, DH), lambda j: (0, j, 0))],
        out_shape=[jax.ShapeDtypeStruct((NP, D), jnp.float32),
                   jax.ShapeDtypeStruct((NC, NP, DH), jnp.float32)],
    )(p, dis)


def _tc_mm1(u0, u1, p, dis, W, b):
    def body(u0r, u1r, pr, dr, wr, br, h_ref, g_ref):
        d1 = dr[...][:, 0:1]
        a = pr[...]
        pfull = jnp.concatenate([a[0], a[1]], axis=-1)
        u0v = u0r[...]
        u2 = -2.0 * d1 * pfull - u0v
        w = wr[...]
        acc = (jnp.dot(u0v, w[0], preferred_element_type=jnp.float32,
                       precision=lax.Precision.HIGHEST)
               + jnp.dot(u1r[...], w[1], preferred_element_type=jnp.float32,
                         precision=lax.Precision.HIGHEST)
               + jnp.dot(u2, w[2], preferred_element_type=jnp.float32,
                         precision=lax.Precision.HIGHEST)
               + br[...])
        h = jnp.maximum(acc, 0.0)
        h_ref[...] = h
        g = d1 * h
        g_ref[0] = g[:, :DH]
        g_ref[1] = g[:, DH:]

    return pl.pallas_call(
        body,
        grid=(GRID,),
        in_specs=[pl.BlockSpec((BN, D), lambda j: (j, 0)),
                  pl.BlockSpec((BN, D), lambda j: (j, 0)),
                  pl.BlockSpec((NC, BN, DH), lambda j: (0, j, 0)),
                  pl.BlockSpec((BN, 16), lambda j: (j, 0)),
                  pl.BlockSpec((3, D, D), lambda j: (0, 0, 0)),
                  pl.BlockSpec((1, D), lambda j: (0, 0))],
        out_specs=[pl.BlockSpec((BN, D), lambda j: (j, 0)),
                   pl.BlockSpec((NC, BN, DH), lambda j: (0, j, 0))],
        out_shape=[jax.ShapeDtypeStruct((NP, D), jnp.float32),
                   jax.ShapeDtypeStruct((NC, NP, DH), jnp.float32)],
    )(u0, u1, p, dis, W, b)


def _tc_mm2(u0, u1, p, dis, W, b):
    def body(u0r, u1r, pr, dr, wr, br, o_ref):
        d1 = dr[...][:, 0:1]
        a = pr[...]
        pfull = jnp.concatenate([a[0], a[1]], axis=-1)
        u0v = u0r[...]
        u2 = -2.0 * d1 * pfull - u0v
        w = wr[...]
        acc = (jnp.dot(u0v, w[0], preferred_element_type=jnp.float32,
                       precision=lax.Precision.HIGHEST)
               + jnp.dot(u1r[...], w[1], preferred_element_type=jnp.float32,
                         precision=lax.Precision.HIGHEST)
               + jnp.dot(u2, w[2], preferred_element_type=jnp.float32,
                         precision=lax.Precision.HIGHEST)
               + br[...])
        m = jnp.max(acc, axis=-1, keepdims=True)
        sh = acc - m
        lse = jnp.log(jnp.sum(jnp.exp(sh), axis=-1, keepdims=True))
        o_ref[...] = sh - lse

    return pl.pallas_call(
        body,
        grid=(GRID,),
        in_specs=[pl.BlockSpec((BN, D), lambda j: (j, 0)),
                  pl.BlockSpec((BN, D), lambda j: (j, 0)),
                  pl.BlockSpec((NC, BN, DH), lambda j: (0, j, 0)),
                  pl.BlockSpec((BN, 16), lambda j: (j, 0)),
                  pl.BlockSpec((3, D, CLS), lambda j: (0, 0, 0)),
                  pl.BlockSpec((1, CLS), lambda j: (0, 0))],
        out_specs=pl.BlockSpec((BN, CLS), lambda j: (j, 0)),
        out_shape=jax.ShapeDtypeStruct((NP, CLS), jnp.float32),
    )(u0, u1, p, dis, W, b)


def kernel(x, edge_index, W1, b1, W2, b2):
    xp = jnp.pad(x, ((0, NP - N), (0, 0)))
    rowm = edge_index[0].reshape(NS, NCH, 1, CHUNK)
    colm = edge_index[1].reshape(NS, NCH, 1, CHUNK)
    ei = jnp.concatenate([rowm, colm], axis=2)
    ei_deg = jnp.concatenate([jnp.zeros_like(rowm), rowm], axis=2)
    ones_tbl = jnp.ones((NC, 8, DH), jnp.float32)
    b1r = b1.reshape(1, D)
    b2r = b2.reshape(1, CLS)

    deg_part = _sc_prop(ones_tbl, ei_deg)
    dis, g0 = _tc_dis_g0(deg_part, xp)

    p1 = _sc_prop(g0, ei)
    u1, g1 = _tc_fuse(p1, dis)
    p2 = _sc_prop(g1, ei)
    h, g2 = _tc_mm1(xp, u1, p2, dis, W1, b1r)

    p3 = _sc_prop(g2, ei)
    v1, g3 = _tc_fuse(p3, dis)
    p4 = _sc_prop(g3, ei)
    return _tc_mm2(h, v1, p4, dis, W2, b2r)[:N]


# speedup vs baseline: 5.9453x; 3.0674x over previous
import functools

import jax
import jax.numpy as jnp
from jax import lax
from jax.experimental import pallas as pl
from jax.experimental.pallas import tpu as pltpu
from jax.experimental.pallas import tpu_sc as plsc

N = 10000
NP = 10240
D = 128
DH = 64
E = 320000
CLS = 40
NC = 2
NS = 16
EW = E // NS
CHUNK = 80
NCH = EW // CHUNK
RPT = NP // NS
NBUF = 5
LOOK = 2

BN = 1024
GRID = NP // BN

_mesh = plsc.VectorSubcoreMesh(core_axis_name="c", subcore_axis_name="s")


def _fill_zero(ref, rows, width):
    @pl.loop(0, rows)
    def _(r):
        @pl.loop(0, width, step=16)
        def _(cc):
            ref.at[pl.ds(r, 1), pl.ds(cc, 16)][...] = jnp.zeros((1, 16), jnp.float32)


def _prop_body(g2_hbm, ei_hbm, out_hbm, idx_v, rows_v, acc_sh, sem_i, sem_g, sem_s):
    c = lax.axis_index("c")
    s = lax.axis_index("s")

    _fill_zero(rows_v.at[0], CHUNK, DH)

    @pl.loop(0, RPT // CHUNK)
    def _(k):
        pltpu.sync_copy(rows_v.at[0], acc_sh.at[pl.ds(s * RPT + k * CHUNK, CHUNK)])

    plsc.subcore_barrier()

    for b in range(NBUF):
        pltpu.async_copy(ei_hbm.at[s, b], idx_v.at[b], sem_i.at[b])

    @pl.loop(0, NCH // NBUF)
    def _(o):
        j0 = o * NBUF
        for b in range(NBUF):
            j = j0 + b
            pltpu.make_async_copy(ei_hbm.at[s, 0], idx_v.at[b],
                                  sem_i.at[b]).wait()
            pltpu.async_copy(g2_hbm.at[c].at[idx_v.at[b].at[0]], rows_v.at[b],
                             sem_g.at[b])

            b2 = (b - LOOK) % NBUF

            @pl.when(j >= LOOK)
            def _():
                pltpu.make_async_copy(g2_hbm.at[0].at[idx_v.at[0].at[0]],
                                      rows_v.at[b2], sem_g.at[b2]).wait()
                pltpu.async_copy(rows_v.at[b2], acc_sh.at[idx_v.at[b2].at[1]],
                                 sem_s.at[b2], add=True)
                pltpu.make_async_copy(rows_v.at[b2],
                                      acc_sh.at[idx_v.at[b2].at[1]],
                                      sem_s.at[b2]).wait()

                @pl.when(j - LOOK + NBUF < NCH)
                def _():
                    pltpu.async_copy(ei_hbm.at[s, j - LOOK + NBUF],
                                     idx_v.at[b2], sem_i.at[b2])

    for k in range(NCH - LOOK, NCH):
        b2 = k % NBUF
        pltpu.make_async_copy(g2_hbm.at[0].at[idx_v.at[0].at[0]],
                              rows_v.at[b2], sem_g.at[b2]).wait()
        pltpu.async_copy(rows_v.at[b2], acc_sh.at[idx_v.at[b2].at[1]],
                         sem_s.at[b2], add=True)
        pltpu.make_async_copy(rows_v.at[b2], acc_sh.at[idx_v.at[b2].at[1]],
                              sem_s.at[b2]).wait()

    plsc.subcore_barrier()
    pltpu.sync_copy(acc_sh.at[pl.ds(s * RPT, RPT)],
                    out_hbm.at[c, pl.ds(s * RPT, RPT)])


def _sc_prop(g2, ei):
    return pl.kernel(
        _prop_body,
        out_type=jax.ShapeDtypeStruct((NC, NP, DH), jnp.float32),
        mesh=_mesh,
        compiler_params=pltpu.CompilerParams(use_tc_tiling_on_sc=False),
        scratch_types=[
            pltpu.VMEM((NBUF, 2, CHUNK), jnp.int32),
            pltpu.VMEM((NBUF, CHUNK, DH), jnp.float32),
            pltpu.VMEM_SHARED((NP, DH), jnp.float32),
            pltpu.SemaphoreType.DMA((NBUF,)),
            pltpu.SemaphoreType.DMA((NBUF,)),
            pltpu.SemaphoreType.DMA((NBUF,)),
        ],
    )(g2, ei)


def _tc_dis_g0(deg_part, x):
    def body(dp, xr, dis_ref, g0_ref):
        d = dp[...][0, :, 0:1]
        dis1 = jnp.where(d > 0, lax.rsqrt(d), 0.0)
        dis_ref[...] = jnp.broadcast_to(dis1, (BN, 16))
        xv = xr[...]
        g0_ref[0] = xv[:, :DH] * dis1
        g0_ref[1] = xv[:, DH:] * dis1

    return pl.pallas_call(
        body,
        grid=(GRID,),
        in_specs=[pl.BlockSpec((NC, BN, DH), lambda j: (0, j, 0)),
                  pl.BlockSpec((BN, D), lambda j: (j, 0))],
        out_specs=[pl.BlockSpec((BN, 16), lambda j: (j, 0)),
                   pl.BlockSpec((NC, BN, DH), lambda j: (0, j, 0))],
        out_shape=[jax.ShapeDtypeStruct((NP, 16), jnp.float32),
                   jax.ShapeDtypeStruct((NC, NP, DH), jnp.float32)],
    )(deg_part, x)


def _tc_fuse(p, dis):
    def body(pr, dr, u_ref, g_ref):
        d1 = dr[...][:, 0:1]
        a = pr[...]
        pfull = jnp.concatenate([a[0], a[1]], axis=-1)
        u = -d1 * pfull
        u_ref[...] = u
        g = d1 * u
        g_ref[0] = g[:, :DH]
        g_ref[1] = g[:, DH:]

    return pl.pallas_call(
        body,
        grid=(GRID,),
        in_specs=[pl.BlockSpec((NC, BN, DH), lambda j: (0, j, 0)),
                  pl.BlockSpec((BN, 16), lambda j: (j, 0))],
        out_specs=[pl.BlockSpec((BN, D), lambda j: (j, 0)),
                   pl.BlockSpec((NC, BN, DH), lambda j: (0, j, 0))],
        out_shape=[jax.ShapeDtypeStruct((NP, D), jnp.float32),
                   jax.ShapeDtypeStruct((NC, NP, DH), jnp.float32)],
    )(p, dis)


def _tc_mm1(u0, u1, p, dis, W, b):
    def body(u0r, u1r, pr, dr, wr, br, h_ref, g_ref):
        d1 = dr[...][:, 0:1]
        a = pr[...]
        pfull = jnp.concatenate([a[0], a[1]], axis=-1)
        u0v = u0r[...]
        u2 = -2.0 * d1 * pfull - u0v
        w = wr[...]
        acc = (jnp.dot(u0v, w[0], preferred_element_type=jnp.float32,
                       precision=lax.Precision.HIGHEST)
               + jnp.dot(u1r[...], w[1], preferred_element_type=jnp.float32,
                         precision=lax.Precision.HIGHEST)
               + jnp.dot(u2, w[2], preferred_element_type=jnp.float32,
                         precision=lax.Precision.HIGHEST)
               + br[...])
        h = jnp.maximum(acc, 0.0)
        h_ref[...] = h
        g = d1 * h
        g_ref[0] = g[:, :DH]
        g_ref[1] = g[:, DH:]

    return pl.pallas_call(
        body,
        grid=(GRID,),
        in_specs=[pl.BlockSpec((BN, D), lambda j: (j, 0)),
                  pl.BlockSpec((BN, D), lambda j: (j, 0)),
                  pl.BlockSpec((NC, BN, DH), lambda j: (0, j, 0)),
                  pl.BlockSpec((BN, 16), lambda j: (j, 0)),
                  pl.BlockSpec((3, D, D), lambda j: (0, 0, 0)),
                  pl.BlockSpec((1, D), lambda j: (0, 0))],
        out_specs=[pl.BlockSpec((BN, D), lambda j: (j, 0)),
                   pl.BlockSpec((NC, BN, DH), lambda j: (0, j, 0))],
        out_shape=[jax.ShapeDtypeStruct((NP, D), jnp.float32),
                   jax.ShapeDtypeStruct((NC, NP, DH), jnp.float32)],
    )(u0, u1, p, dis, W, b)


def _tc_mm2(u0, u1, p, dis, W, b):
    def body(u0r, u1r, pr, dr, wr, br, o_ref):
        d1 = dr[...][:, 0:1]
        a = pr[...]
        pfull = jnp.concatenate([a[0], a[1]], axis=-1)
        u0v = u0r[...]
        u2 = -2.0 * d1 * pfull - u0v
        w = wr[...]
        acc = (jnp.dot(u0v, w[0], preferred_element_type=jnp.float32,
                       precision=lax.Precision.HIGHEST)
               + jnp.dot(u1r[...], w[1], preferred_element_type=jnp.float32,
                         precision=lax.Precision.HIGHEST)
               + jnp.dot(u2, w[2], preferred_element_type=jnp.float32,
                         precision=lax.Precision.HIGHEST)
               + br[...])
        m = jnp.max(acc, axis=-1, keepdims=True)
        sh = acc - m
        lse = jnp.log(jnp.sum(jnp.exp(sh), axis=-1, keepdims=True))
        o_ref[...] = sh - lse

    return pl.pallas_call(
        body,
        grid=(GRID,),
        in_specs=[pl.BlockSpec((BN, D), lambda j: (j, 0)),
                  pl.BlockSpec((BN, D), lambda j: (j, 0)),
                  pl.BlockSpec((NC, BN, DH), lambda j: (0, j, 0)),
                  pl.BlockSpec((BN, 16), lambda j: (j, 0)),
                  pl.BlockSpec((3, D, CLS), lambda j: (0, 0, 0)),
                  pl.BlockSpec((1, CLS), lambda j: (0, 0))],
        out_specs=pl.BlockSpec((BN, CLS), lambda j: (j, 0)),
        out_shape=jax.ShapeDtypeStruct((NP, CLS), jnp.float32),
    )(u0, u1, p, dis, W, b)


def kernel(x, edge_index, W1, b1, W2, b2):
    xp = jnp.pad(x, ((0, NP - N), (0, 0)))
    rowm = edge_index[0].reshape(NS, NCH, 1, CHUNK)
    colm = edge_index[1].reshape(NS, NCH, 1, CHUNK)
    ei = jnp.concatenate([rowm, colm], axis=2)
    idx8 = jnp.tile(jnp.arange(8, dtype=edge_index.dtype), E // 8).reshape(
        NS, NCH, 1, CHUNK)
    ei_deg = jnp.concatenate([idx8, rowm], axis=2)
    ones_tbl = jnp.ones((NC, 8, DH), jnp.float32)
    b1r = b1.reshape(1, D)
    b2r = b2.reshape(1, CLS)

    deg_part = _sc_prop(ones_tbl, ei_deg)
    dis, g0 = _tc_dis_g0(deg_part, xp)

    p1 = _sc_prop(g0, ei)
    u1, g1 = _tc_fuse(p1, dis)
    p2 = _sc_prop(g1, ei)
    h, g2 = _tc_mm1(xp, u1, p2, dis, W1, b1r)

    p3 = _sc_prop(g2, ei)
    v1, g3 = _tc_fuse(p3, dis)
    p4 = _sc_prop(g3, ei)
    return _tc_mm2(h, v1, p4, dis, W2, b2r)[:N]


# speedup vs baseline: 16.8768x; 2.8387x over previous
import functools

import jax
import jax.numpy as jnp
from jax import lax
from jax.experimental import pallas as pl
from jax.experimental.pallas import tpu as pltpu
from jax.experimental.pallas import tpu_sc as plsc

N = 10000
NP = 10240
D = 128
DH = 64
E = 320000
CLS = 40
NC = 2
NS = 16
EW = E // NS
CHUNK = 80
NCH = EW // CHUNK
RPT = NP // NS
NBUF = 5
LOOK = 2

BN = 1024
GRID = NP // BN

_mesh = plsc.VectorSubcoreMesh(core_axis_name="c", subcore_axis_name="s")


def _fill_zero(ref, rows, width):
    @pl.loop(0, rows)
    def _(r):
        @pl.loop(0, width, step=16)
        def _(cc):
            ref.at[pl.ds(r, 1), pl.ds(cc, 16)][...] = jnp.zeros((1, 16), jnp.float32)


def _prop_body(g2_hbm, ei_hbm, out_hbm, idx_v, rows_v, acc_sh, sem_i, sem_g, sem_s):
    c = lax.axis_index("c")
    s = lax.axis_index("s")

    _fill_zero(rows_v.at[0], CHUNK, DH)

    @pl.loop(0, RPT // CHUNK)
    def _(k):
        pltpu.sync_copy(rows_v.at[0], acc_sh.at[pl.ds(s * RPT + k * CHUNK, CHUNK)])

    plsc.subcore_barrier()

    for b in range(NBUF):
        pltpu.async_copy(ei_hbm.at[s, b], idx_v.at[b], sem_i.at[b])

    @pl.loop(0, NCH // NBUF)
    def _(o):
        j0 = o * NBUF
        for b in range(NBUF):
            j = j0 + b
            pltpu.make_async_copy(ei_hbm.at[s, 0], idx_v.at[b],
                                  sem_i.at[b]).wait()
            pltpu.async_copy(g2_hbm.at[c].at[idx_v.at[b].at[0]], rows_v.at[b],
                             sem_g.at[b])

            b2 = (b - LOOK) % NBUF

            @pl.when(j >= LOOK)
            def _():
                pltpu.make_async_copy(g2_hbm.at[0].at[idx_v.at[0].at[0]],
                                      rows_v.at[b2], sem_g.at[b2]).wait()
                pltpu.async_copy(rows_v.at[b2], acc_sh.at[idx_v.at[b2].at[1]],
                                 sem_s.at[b2], add=True)
                pltpu.make_async_copy(rows_v.at[b2],
                                      acc_sh.at[idx_v.at[b2].at[1]],
                                      sem_s.at[b2]).wait()

                @pl.when(j - LOOK + NBUF < NCH)
                def _():
                    pltpu.async_copy(ei_hbm.at[s, j - LOOK + NBUF],
                                     idx_v.at[b2], sem_i.at[b2])

    for k in range(NCH - LOOK, NCH):
        b2 = k % NBUF
        pltpu.make_async_copy(g2_hbm.at[0].at[idx_v.at[0].at[0]],
                              rows_v.at[b2], sem_g.at[b2]).wait()
        pltpu.async_copy(rows_v.at[b2], acc_sh.at[idx_v.at[b2].at[1]],
                         sem_s.at[b2], add=True)
        pltpu.make_async_copy(rows_v.at[b2], acc_sh.at[idx_v.at[b2].at[1]],
                              sem_s.at[b2]).wait()

    plsc.subcore_barrier()
    pltpu.sync_copy(acc_sh.at[pl.ds(s * RPT, RPT)],
                    out_hbm.at[c, pl.ds(s * RPT, RPT)])


def _sc_prop(g2, ei):
    return pl.kernel(
        _prop_body,
        out_type=jax.ShapeDtypeStruct((NC, NP, DH), jnp.float32),
        mesh=_mesh,
        compiler_params=pltpu.CompilerParams(use_tc_tiling_on_sc=False),
        scratch_types=[
            pltpu.VMEM((NBUF, 2, CHUNK), jnp.int32),
            pltpu.VMEM((NBUF, CHUNK, DH), jnp.float32),
            pltpu.VMEM_SHARED((NP, DH), jnp.float32),
            pltpu.SemaphoreType.DMA((NBUF,)),
            pltpu.SemaphoreType.DMA((NBUF,)),
            pltpu.SemaphoreType.DMA((NBUF,)),
        ],
    )(g2, ei)


def _tc_dis_g0(deg_part, x):
    def body(dp, xr, dis_ref, g0_ref):
        d = dp[...][0, :, 0:1]
        dis1 = jnp.where(d > 0, lax.rsqrt(d), 0.0)
        dis_ref[...] = jnp.broadcast_to(dis1, (BN, 16))
        xv = xr[...]
        g0_ref[0] = xv[:, :DH] * dis1
        g0_ref[1] = xv[:, DH:] * dis1

    return pl.pallas_call(
        body,
        grid=(GRID,),
        in_specs=[pl.BlockSpec((NC, BN, DH), lambda j: (0, j, 0)),
                  pl.BlockSpec((BN, D), lambda j: (j, 0))],
        out_specs=[pl.BlockSpec((BN, 16), lambda j: (j, 0)),
                   pl.BlockSpec((NC, BN, DH), lambda j: (0, j, 0))],
        out_shape=[jax.ShapeDtypeStruct((NP, 16), jnp.float32),
                   jax.ShapeDtypeStruct((NC, NP, DH), jnp.float32)],
    )(deg_part, x)


def _tc_fuse(p, dis):
    def body(pr, dr, u_ref, g_ref):
        d1 = dr[...][:, 0:1]
        a = pr[...]
        pfull = jnp.concatenate([a[0], a[1]], axis=-1)
        u = -d1 * pfull
        u_ref[...] = u
        g = d1 * u
        g_ref[0] = g[:, :DH]
        g_ref[1] = g[:, DH:]

    return pl.pallas_call(
        body,
        grid=(GRID,),
        in_specs=[pl.BlockSpec((NC, BN, DH), lambda j: (0, j, 0)),
                  pl.BlockSpec((BN, 16), lambda j: (j, 0))],
        out_specs=[pl.BlockSpec((BN, D), lambda j: (j, 0)),
                   pl.BlockSpec((NC, BN, DH), lambda j: (0, j, 0))],
        out_shape=[jax.ShapeDtypeStruct((NP, D), jnp.float32),
                   jax.ShapeDtypeStruct((NC, NP, DH), jnp.float32)],
    )(p, dis)


def _tc_mm1(u0, u1, p, dis, W, b):
    def body(u0r, u1r, pr, dr, wr, br, h_ref, g_ref):
        d1 = dr[...][:, 0:1]
        a = pr[...]
        pfull = jnp.concatenate([a[0], a[1]], axis=-1)
        u0v = u0r[...]
        u2 = -2.0 * d1 * pfull - u0v
        w = wr[...]
        acc = (jnp.dot(u0v, w[0], preferred_element_type=jnp.float32,
                       precision=lax.Precision.HIGHEST)
               + jnp.dot(u1r[...], w[1], preferred_element_type=jnp.float32,
                         precision=lax.Precision.HIGHEST)
               + jnp.dot(u2, w[2], preferred_element_type=jnp.float32,
                         precision=lax.Precision.HIGHEST)
               + br[...])
        h = jnp.maximum(acc, 0.0)
        h_ref[...] = h
        g = d1 * h
        g_ref[0] = g[:, :DH]
        g_ref[1] = g[:, DH:]

    return pl.pallas_call(
        body,
        grid=(GRID,),
        in_specs=[pl.BlockSpec((BN, D), lambda j: (j, 0)),
                  pl.BlockSpec((BN, D), lambda j: (j, 0)),
                  pl.BlockSpec((NC, BN, DH), lambda j: (0, j, 0)),
                  pl.BlockSpec((BN, 16), lambda j: (j, 0)),
                  pl.BlockSpec((3, D, D), lambda j: (0, 0, 0)),
                  pl.BlockSpec((1, D), lambda j: (0, 0))],
        out_specs=[pl.BlockSpec((BN, D), lambda j: (j, 0)),
                   pl.BlockSpec((NC, BN, DH), lambda j: (0, j, 0))],
        out_shape=[jax.ShapeDtypeStruct((NP, D), jnp.float32),
                   jax.ShapeDtypeStruct((NC, NP, DH), jnp.float32)],
    )(u0, u1, p, dis, W, b)


def _tc_mm2(u0, u1, p, dis, W, b):
    def body(u0r, u1r, pr, dr, wr, br, o_ref):
        d1 = dr[...][:, 0:1]
        a = pr[...]
        pfull = jnp.concatenate([a[0], a[1]], axis=-1)
        u0v = u0r[...]
        u2 = -2.0 * d1 * pfull - u0v
        w = wr[...]
        acc = (jnp.dot(u0v, w[0], preferred_element_type=jnp.float32,
                       precision=lax.Precision.HIGHEST)
               + jnp.dot(u1r[...], w[1], preferred_element_type=jnp.float32,
                         precision=lax.Precision.HIGHEST)
               + jnp.dot(u2, w[2], preferred_element_type=jnp.float32,
                         precision=lax.Precision.HIGHEST)
               + br[...])
        m = jnp.max(acc, axis=-1, keepdims=True)
        sh = acc - m
        lse = jnp.log(jnp.sum(jnp.exp(sh), axis=-1, keepdims=True))
        o_ref[...] = sh - lse

    return pl.pallas_call(
        body,
        grid=(GRID,),
        in_specs=[pl.BlockSpec((BN, D), lambda j: (j, 0)),
                  pl.BlockSpec((BN, D), lambda j: (j, 0)),
                  pl.BlockSpec((NC, BN, DH), lambda j: (0, j, 0)),
                  pl.BlockSpec((BN, 16), lambda j: (j, 0)),
                  pl.BlockSpec((3, D, CLS), lambda j: (0, 0, 0)),
                  pl.BlockSpec((1, CLS), lambda j: (0, 0))],
        out_specs=pl.BlockSpec((BN, CLS), lambda j: (j, 0)),
        out_shape=jax.ShapeDtypeStruct((NP, CLS), jnp.float32),
    )(u0, u1, p, dis, W, b)


def kernel(x, edge_index, W1, b1, W2, b2):
    xp = jnp.pad(x, ((0, NP - N), (0, 0)))
    rowm = edge_index[0].reshape(NS, NCH, 1, CHUNK)
    colm = edge_index[1].reshape(NS, NCH, 1, CHUNK)
    ei = jnp.concatenate([rowm, colm], axis=2)
    ei_deg = jnp.concatenate([rowm, rowm], axis=2)
    ones_tbl = jnp.ones((NC, NP, DH), jnp.float32)
    b1r = b1.reshape(1, D)
    b2r = b2.reshape(1, CLS)

    deg_part = _sc_prop(ones_tbl, ei_deg)
    dis, g0 = _tc_dis_g0(deg_part, xp)

    p1 = _sc_prop(g0, ei)
    u1, g1 = _tc_fuse(p1, dis)
    p2 = _sc_prop(g1, ei)
    h, g2 = _tc_mm1(xp, u1, p2, dis, W1, b1r)

    p3 = _sc_prop(g2, ei)
    v1, g3 = _tc_fuse(p3, dis)
    p4 = _sc_prop(g3, ei)
    return _tc_mm2(h, v1, p4, dis, W2, b2r)[:N]
